# baseline (speedup 1.0000x reference)
import functools

import jax
import jax.numpy as jnp
from jax import lax
from jax.experimental import pallas as pl
from jax.experimental.pallas import tpu as pltpu
from jax.experimental.pallas import tpu_sc as plsc

NC = 2
NS = 16
L = 16
NW = NC * NS

Q = 1024
K = 100000
CHUNK = ((K + NW * L - 1) // (NW * L)) * L
KPAD = CHUNK * NW
QPW = Q // NW
UNROLL = 16

_mesh = functools.partial(
    plsc.VectorSubcoreMesh, core_axis_name="c", subcore_axis_name="s")


@functools.partial(
    pl.kernel,
    out_type=(
        jax.ShapeDtypeStruct((NW, Q), jnp.float32),
        jax.ShapeDtypeStruct((NW, Q), jnp.int32),
    ),
    mesh=_mesh(),
    scratch_types=[
        pltpu.VMEM((CHUNK,), jnp.float32),
        pltpu.VMEM((CHUNK,), jnp.float32),
        pltpu.VMEM((Q,), jnp.float32),
        pltpu.VMEM((Q,), jnp.float32),
        pltpu.VMEM((Q,), jnp.float32),
        pltpu.VMEM((Q,), jnp.int32),
    ],
)
def _stage1(qx_hbm, qy_hbm, mx_hbm, my_hbm, outd_hbm, outi_hbm,
            mxv, myv, qxv, qyv, bdv, biv):
    c = lax.axis_index("c")
    s = lax.axis_index("s")
    w = s * NC + c
    base = w * CHUNK

    pltpu.sync_copy(mx_hbm.at[pl.ds(base, CHUNK)], mxv)
    pltpu.sync_copy(my_hbm.at[pl.ds(base, CHUNK)], myv)
    pltpu.sync_copy(qx_hbm, qxv)
    pltpu.sync_copy(qy_hbm, qyv)

    def per_group(g, _):
        qxb = qxv[pl.ds(g * L, L)]
        qyb = qyv[pl.ds(g * L, L)]

        def inner(i, carry):
            bd, bi = carry
            k0 = i * UNROLL
            mxvec = mxv[pl.ds(k0, UNROLL)]
            myvec = myv[pl.ds(k0, UNROLL)]
            for u in range(UNROLL):
                k = k0 + u
                mxb = jnp.full((L,), mxvec[u], jnp.float32)
                myb = jnp.full((L,), myvec[u], jnp.float32)
                dx = qxb - mxb
                dy = qyb - myb
                d = dx * dx + dy * dy
                lt = d < bd
                bd = jnp.where(lt, d, bd)
                bi = jnp.where(lt, base + k, bi)
            return bd, bi

        bd0 = jnp.full((L,), jnp.inf, jnp.float32)
        bi0 = jnp.zeros((L,), jnp.int32)
        bd, bi = lax.fori_loop(0, CHUNK // UNROLL, inner, (bd0, bi0))
        bdv[pl.ds(g * L, L)] = bd
        biv[pl.ds(g * L, L)] = bi
        return 0

    lax.fori_loop(0, Q // L, per_group, 0)

    pltpu.sync_copy(bdv, outd_hbm.at[w])
    pltpu.sync_copy(biv, outi_hbm.at[w])


@functools.partial(
    pl.kernel,
    out_type=(
        jax.ShapeDtypeStruct((Q,), jnp.float32),
        jax.ShapeDtypeStruct((Q,), jnp.float32),
        jax.ShapeDtypeStruct((Q,), jnp.float32),
    ),
    mesh=_mesh(),
    scratch_types=[
        pltpu.VMEM((NW, QPW), jnp.float32),
        pltpu.VMEM((NW, QPW), jnp.int32),
        pltpu.VMEM((QPW,), jnp.int32),
        pltpu.VMEM((QPW,), jnp.float32),
        pltpu.VMEM((QPW,), jnp.float32),
        pltpu.VMEM((QPW,), jnp.float32),
        pltpu.SemaphoreType.DMA,
    ],
)
def _stage2(dall_hbm, iall_hbm, mx_hbm, my_hbm, ts_hbm,
            px_hbm, py_hbm, lin_hbm,
            dbuf, ibuf, biv, pxv, pyv, linv, sem):
    c = lax.axis_index("c")
    s = lax.axis_index("s")
    w = s * NC + c
    qbase = w * QPW

    copies = []
    for r in range(NW):
        copies.append(
            pltpu.async_copy(dall_hbm.at[r, pl.ds(qbase, QPW)], dbuf.at[r], sem))
        copies.append(
            pltpu.async_copy(iall_hbm.at[r, pl.ds(qbase, QPW)], ibuf.at[r], sem))
    for cp in copies:
        cp.wait()

    for j in range(QPW // L):
        bd = jnp.full((L,), jnp.inf, jnp.float32)
        bi = jnp.zeros((L,), jnp.int32)
        for r in range(NW):
            dr = dbuf[r, pl.ds(j * L, L)]
            ir = ibuf[r, pl.ds(j * L, L)]
            lt = dr < bd
            bd = jnp.where(lt, dr, bd)
            bi = jnp.where(lt, ir, bi)
        biv[pl.ds(j * L, L)] = bi

    pltpu.async_copy(mx_hbm.at[biv], pxv, sem).wait()
    pltpu.async_copy(my_hbm.at[biv], pyv, sem).wait()
    pltpu.async_copy(ts_hbm.at[biv], linv, sem).wait()

    pltpu.sync_copy(pxv, px_hbm.at[pl.ds(qbase, QPW)])
    pltpu.sync_copy(pyv, py_hbm.at[pl.ds(qbase, QPW)])
    pltpu.sync_copy(linv, lin_hbm.at[pl.ds(qbase, QPW)])


@jax.jit
def kernel(euclidean_data, maze_points, ts_proj):
    ed = euclidean_data.astype(maze_points.dtype)
    qx = ed[:, 0]
    qy = ed[:, 1]
    pad = KPAD - K
    mxp = jnp.concatenate([maze_points[:, 0], jnp.full((pad,), 1e6, jnp.float32)])
    myp = jnp.concatenate([maze_points[:, 1], jnp.full((pad,), 1e6, jnp.float32)])
    dall, iall = _stage1(qx, qy, mxp, myp)
    px, py, lin = _stage2(dall, iall, mxp, myp, ts_proj)
    projected = jnp.stack([px, py], axis=-1)
    return projected, lin


# speedup vs baseline: 1.6187x; 1.6187x over previous
import functools

import jax
import jax.numpy as jnp
from jax import lax
from jax.experimental import pallas as pl
from jax.experimental.pallas import tpu as pltpu
from jax.experimental.pallas import tpu_sc as plsc

NC = 2
NS = 16
L = 16
NW = NC * NS

Q = 1024
K = 100000

T_TC = 73216
SC_N = K - T_TC
CHUNK = ((SC_N + NW * L - 1) // (NW * L)) * L
SC_KPAD = CHUNK * NW
KG = T_TC + SC_KPAD

QPW = Q // NW
UNROLL = 16
KUNROLL = 4
TKB = T_TC // 128
IMAX = 0x7FFFFFFF

_mesh = functools.partial(
    plsc.VectorSubcoreMesh, core_axis_name="c", subcore_axis_name="s")


def _tc_body(qx_ref, qy_ref, mx_ref, my_ref, outd_ref, outi_ref):
    lane_iota = lax.broadcasted_iota(jnp.int32, (8, 128), 1)
    inf8 = jnp.full((8, 128), jnp.inf, jnp.float32)
    zero8 = jnp.zeros((8, 128), jnp.int32)

    def per_qblock(qb, acc):
        accd, acci = acc
        qxb = jnp.broadcast_to(qx_ref[pl.ds(qb * 8, 8), :], (8, 128))
        qyb = jnp.broadcast_to(qy_ref[pl.ds(qb * 8, 8), :], (8, 128))

        def inner(t, carry):
            bd, bi = carry
            for u in range(KUNROLL):
                kb = t * KUNROLL + u
                mxb = jnp.broadcast_to(mx_ref[pl.ds(kb, 1), :], (8, 128))
                myb = jnp.broadcast_to(my_ref[pl.ds(kb, 1), :], (8, 128))
                dx = qxb - mxb
                dy = qyb - myb
                d = dx * dx + dy * dy
                lt = d < bd
                bd = jnp.where(lt, d, bd)
                bi = jnp.where(lt, lane_iota + kb * 128, bi)
            return bd, bi

        bd, bi = lax.fori_loop(0, TKB // KUNROLL, inner, (inf8, zero8))
        mind = jnp.min(bd, axis=1, keepdims=True)
        bim = jnp.where(bd == mind, bi, IMAX)
        mini = jnp.min(bim, axis=1, keepdims=True)
        sel = lane_iota == qb
        accd = jnp.where(sel, jnp.broadcast_to(mind, (8, 128)), accd)
        acci = jnp.where(sel, jnp.broadcast_to(mini, (8, 128)), acci)
        return accd, acci

    accd, acci = lax.fori_loop(0, Q // 8, per_qblock, (inf8, zero8))
    outd_ref[:, :] = accd
    outi_ref[:, :] = acci


_tc_stage = pl.pallas_call(
    _tc_body,
    out_shape=(
        jax.ShapeDtypeStruct((8, Q // 8), jnp.float32),
        jax.ShapeDtypeStruct((8, Q // 8), jnp.int32),
    ),
)


@functools.partial(
    pl.kernel,
    out_type=(
        jax.ShapeDtypeStruct((NW, Q), jnp.float32),
        jax.ShapeDtypeStruct((NW, Q), jnp.int32),
    ),
    mesh=_mesh(),
    scratch_types=[
        pltpu.VMEM((CHUNK,), jnp.float32),
        pltpu.VMEM((CHUNK,), jnp.float32),
        pltpu.VMEM((Q,), jnp.float32),
        pltpu.VMEM((Q,), jnp.float32),
        pltpu.VMEM((Q,), jnp.float32),
        pltpu.VMEM((Q,), jnp.int32),
    ],
)
def _stage1_sc(qx_hbm, qy_hbm, mx_hbm, my_hbm, outd_hbm, outi_hbm,
               mxv, myv, qxv, qyv, bdv, biv):
    c = lax.axis_index("c")
    s = lax.axis_index("s")
    w = s * NC + c
    base = w * CHUNK

    pltpu.sync_copy(mx_hbm.at[pl.ds(base, CHUNK)], mxv)
    pltpu.sync_copy(my_hbm.at[pl.ds(base, CHUNK)], myv)
    pltpu.sync_copy(qx_hbm, qxv)
    pltpu.sync_copy(qy_hbm, qyv)

    def per_group(g, _):
        qxb = qxv[pl.ds(g * L, L)]
        qyb = qyv[pl.ds(g * L, L)]

        def inner(i, carry):
            bd, bi = carry
            k0 = i * UNROLL
            mxvec = mxv[pl.ds(k0, UNROLL)]
            myvec = myv[pl.ds(k0, UNROLL)]
            for u in range(UNROLL):
                k = k0 + u
                mxb = jnp.full((L,), mxvec[u], jnp.float32)
                myb = jnp.full((L,), myvec[u], jnp.float32)
                dx = qxb - mxb
                dy = qyb - myb
                d = dx * dx + dy * dy
                lt = d < bd
                bd = jnp.where(lt, d, bd)
                bi = jnp.where(lt, T_TC + base + k, bi)
            return bd, bi

        bd0 = jnp.full((L,), jnp.inf, jnp.float32)
        bi0 = jnp.zeros((L,), jnp.int32)
        bd, bi = lax.fori_loop(0, CHUNK // UNROLL, inner, (bd0, bi0))
        bdv[pl.ds(g * L, L)] = bd
        biv[pl.ds(g * L, L)] = bi
        return 0

    lax.fori_loop(0, Q // L, per_group, 0)

    pltpu.sync_copy(bdv, outd_hbm.at[w])
    pltpu.sync_copy(biv, outi_hbm.at[w])


@functools.partial(
    pl.kernel,
    out_type=(
        jax.ShapeDtypeStruct((Q,), jnp.float32),
        jax.ShapeDtypeStruct((Q,), jnp.float32),
        jax.ShapeDtypeStruct((Q,), jnp.float32),
    ),
    mesh=_mesh(),
    scratch_types=[
        pltpu.VMEM((NW, QPW), jnp.float32),
        pltpu.VMEM((NW, QPW), jnp.int32),
        pltpu.VMEM((QPW,), jnp.float32),
        pltpu.VMEM((QPW,), jnp.int32),
        pltpu.VMEM((QPW,), jnp.int32),
        pltpu.VMEM((QPW,), jnp.float32),
        pltpu.VMEM((QPW,), jnp.float32),
        pltpu.VMEM((QPW,), jnp.float32),
        pltpu.SemaphoreType.DMA,
    ],
)
def _stage2(tcd_hbm, tci_hbm, dall_hbm, iall_hbm, mx_hbm, my_hbm, ts_hbm,
            px_hbm, py_hbm, lin_hbm,
            dbuf, ibuf, tdbuf, tibuf, biv, pxv, pyv, linv, sem):
    c = lax.axis_index("c")
    s = lax.axis_index("s")
    w = s * NC + c
    qbase = w * QPW

    copies = [
        pltpu.async_copy(tcd_hbm.at[pl.ds(qbase, QPW)], tdbuf, sem),
        pltpu.async_copy(tci_hbm.at[pl.ds(qbase, QPW)], tibuf, sem),
    ]
    for r in range(NW):
        copies.append(
            pltpu.async_copy(dall_hbm.at[r, pl.ds(qbase, QPW)], dbuf.at[r], sem))
        copies.append(
            pltpu.async_copy(iall_hbm.at[r, pl.ds(qbase, QPW)], ibuf.at[r], sem))
    for cp in copies:
        cp.wait()

    for j in range(QPW // L):
        bd = tdbuf[pl.ds(j * L, L)]
        bi = tibuf[pl.ds(j * L, L)]
        for r in range(NW):
            dr = dbuf[r, pl.ds(j * L, L)]
            ir = ibuf[r, pl.ds(j * L, L)]
            lt = dr < bd
            bd = jnp.where(lt, dr, bd)
            bi = jnp.where(lt, ir, bi)
        biv[pl.ds(j * L, L)] = bi

    pltpu.async_copy(mx_hbm.at[biv], pxv, sem).wait()
    pltpu.async_copy(my_hbm.at[biv], pyv, sem).wait()
    pltpu.async_copy(ts_hbm.at[biv], linv, sem).wait()

    pltpu.sync_copy(pxv, px_hbm.at[pl.ds(qbase, QPW)])
    pltpu.sync_copy(pyv, py_hbm.at[pl.ds(qbase, QPW)])
    pltpu.sync_copy(linv, lin_hbm.at[pl.ds(qbase, QPW)])


@jax.jit
def kernel(euclidean_data, maze_points, ts_proj):
    ed = euclidean_data.astype(maze_points.dtype)
    qx = ed[:, 0]
    qy = ed[:, 1]
    padv = jnp.full((KG - K,), 1e6, jnp.float32)
    mxg = jnp.concatenate([maze_points[:, 0], padv])
    myg = jnp.concatenate([maze_points[:, 1], padv])

    tc_d2, tc_i2 = _tc_stage(
        qx[:, None], qy[:, None],
        mxg[:T_TC].reshape(TKB, 128), myg[:T_TC].reshape(TKB, 128))
    tc_d = tc_d2.T.reshape(Q)
    tc_i = tc_i2.T.reshape(Q)

    dall, iall = _stage1_sc(qx, qy, mxg[T_TC:], myg[T_TC:])
    px, py, lin = _stage2(tc_d, tc_i, dall, iall, mxg, myg, ts_proj)
    projected = jnp.stack([px, py], axis=-1)
    return projected, lin
